# baseline (speedup 1.0000x reference)
import functools

import jax
import jax.numpy as jnp
from jax import lax
from jax.experimental import pallas as pl
from jax.experimental.pallas import tpu as pltpu
from jax.experimental.pallas import tpu_sc as plsc

NUM_EMB = 1000000
B = 4096
L = 50
D = 32
NC = 2
NS = 16
NW = NC * NS
BPW = B // NW
CROWS = 16
NCHUNK = BPW // CROWS
NBUF = 2
INV_L = 1.0 / L

_mesh = plsc.VectorSubcoreMesh(core_axis_name="c", subcore_axis_name="s")


@functools.partial(
    pl.kernel,
    out_type=jax.ShapeDtypeStruct((B, D), jnp.float32),
    mesh=_mesh,
    scratch_types=[
        pltpu.VMEM((NBUF, L, 4 * D), jnp.float32),
        pltpu.VMEM((BPW, D), jnp.float32),
        pltpu.VMEM_SHARED((NS * BPW * L,), jnp.int32),
        pltpu.SMEM((CROWS * L,), jnp.int32),
        [pltpu.SemaphoreType.DMA] * NBUF,
    ],
)
def _avg_embed(idx_hbm, table_hbm, out_hbm, buf, out_v, idx_sh, idx_s, sems):
    sid = lax.axis_index("s")
    wid = sid * NC + lax.axis_index("c")
    base = wid * BPW

    pltpu.sync_copy(
        idx_hbm.at[pl.ds(wid * BPW * L, BPW * L)],
        idx_sh.at[pl.ds(sid * BPW * L, BPW * L)],
    )

    def fire(i, slot):
        for l in range(L):
            r = idx_s[i * L + l]
            pltpu.async_copy(
                table_hbm.at[pl.ds(r >> 2, 1), :],
                buf.at[slot, pl.ds(l, 1), :],
                sems[slot],
            )

    def drain(slot):
        for l in range(L):
            pltpu.make_async_copy(
                table_hbm.at[pl.ds(0, 1), :],
                buf.at[slot, pl.ds(l, 1), :],
                sems[slot],
            ).wait()

    def reduce(i, row, slot):
        acc0 = jnp.zeros((16,), jnp.float32)
        acc1 = jnp.zeros((16,), jnp.float32)
        for l in range(L):
            sel = idx_s[i * L + l] & 3
            w = [buf[slot, l, pl.ds(q * 16, 16)] for q in range(8)]
            lo = jnp.where(sel < 2,
                           jnp.where(sel == 0, w[0], w[2]),
                           jnp.where(sel == 2, w[4], w[6]))
            hi = jnp.where(sel < 2,
                           jnp.where(sel == 0, w[1], w[3]),
                           jnp.where(sel == 2, w[5], w[7]))
            acc0 = acc0 + lo
            acc1 = acc1 + hi
        out_v[row, pl.ds(0, 16)] = acc0 * INV_L
        out_v[row, pl.ds(16, 16)] = acc1 * INV_L

    def chunk(blk):
        pltpu.sync_copy(
            idx_sh.at[pl.ds(sid * BPW * L + blk * CROWS * L, CROWS * L)], idx_s
        )
        for s in range(NBUF):
            fire(s, s)

        def rows(i):
            for s in range(NBUF):
                drain(s)
                reduce(i + s, blk * CROWS + i + s, s)

                @pl.when(i + s + NBUF < CROWS)
                def _():
                    fire(i + s + NBUF, s)

        pl.loop(0, CROWS, step=NBUF)(rows)

    pl.loop(0, NCHUNK)(chunk)

    pltpu.sync_copy(out_v, out_hbm.at[pl.ds(base, BPW), :])


def kernel(inputs, table):
    return _avg_embed(inputs.reshape(-1), table.reshape(NUM_EMB // 4, 4 * D))


# speedup vs baseline: 1.5485x; 1.5485x over previous
import functools

import jax
import jax.numpy as jnp
from jax import lax
from jax.experimental import pallas as pl
from jax.experimental.pallas import tpu as pltpu
from jax.experimental.pallas import tpu_sc as plsc

B = 4096
L = 50
D = 32
NC = 2
NS = 16
NW = NC * NS
BPW = B // NW
CROWS = 32
NCHUNK = BPW // CROWS
NBUF = 2
INV_L = 1.0 / L

_mesh = plsc.VectorSubcoreMesh(core_axis_name="c", subcore_axis_name="s")


@functools.partial(
    pl.kernel,
    out_type=jax.ShapeDtypeStruct((B, D), jnp.float32),
    mesh=_mesh,
    scratch_types=[
        pltpu.VMEM((NBUF, L, D), jnp.float32),
        pltpu.VMEM((BPW, D), jnp.float32),
        pltpu.VMEM_SHARED((NS * BPW * L,), jnp.int32),
        pltpu.SMEM((CROWS * L,), jnp.int32),
        [pltpu.SemaphoreType.DMA] * NBUF,
    ],
)
def _avg_embed(idx_hbm, table_hbm, out_hbm, buf, out_v, idx_sh, idx_s, sems):
    sid = lax.axis_index("s")
    wid = sid * NC + lax.axis_index("c")
    base = wid * BPW

    pltpu.sync_copy(
        idx_hbm.at[pl.ds(wid * BPW * L, BPW * L)],
        idx_sh.at[pl.ds(sid * BPW * L, BPW * L)],
    )

    def fire(i, slot):
        for l in range(L):
            r = idx_s[i * L + l]
            pltpu.async_copy(
                table_hbm.at[pl.ds(r, 1), :],
                buf.at[slot, pl.ds(l, 1), :],
                sems[slot],
            )

    def drain(slot):
        for l in range(L):
            pltpu.make_async_copy(
                table_hbm.at[pl.ds(0, 1), :],
                buf.at[slot, pl.ds(l, 1), :],
                sems[slot],
            ).wait()

    def reduce(row, slot):
        acc0 = jnp.zeros((16,), jnp.float32)
        acc1 = jnp.zeros((16,), jnp.float32)
        for l in range(L):
            acc0 = acc0 + buf[slot, l, pl.ds(0, 16)]
            acc1 = acc1 + buf[slot, l, pl.ds(16, 16)]
        out_v[row, pl.ds(0, 16)] = acc0 * INV_L
        out_v[row, pl.ds(16, 16)] = acc1 * INV_L

    def chunk(blk):
        pltpu.sync_copy(
            idx_sh.at[pl.ds(sid * BPW * L + blk * CROWS * L, CROWS * L)], idx_s
        )
        for s in range(NBUF):
            fire(s, s)

        def rows(i):
            for s in range(NBUF):
                drain(s)
                reduce(blk * CROWS + i + s, s)

                @pl.when(i + s + NBUF < CROWS)
                def _():
                    fire(i + s + NBUF, s)

        pl.loop(0, CROWS, step=NBUF)(rows)

    pl.loop(0, NCHUNK)(chunk)

    pltpu.sync_copy(out_v, out_hbm.at[pl.ds(base, BPW), :])


def kernel(inputs, table):
    return _avg_embed(inputs.reshape(-1), table)
